# baseline (speedup 1.0000x reference)
import functools

import jax
import jax.numpy as jnp
from jax import lax
from jax.experimental import pallas as pl
from jax.experimental.pallas import tpu as pltpu
from jax.experimental.pallas import tpu_sc as plsc

NC = 2
NS = 16
NW = NC * NS


def _make_sc_edge(n: int, e: int, de: int, chunk: int):
    ew = e // NW
    nrows = n // NS
    mesh = plsc.VectorSubcoreMesh(
        core_axis_name="c", subcore_axis_name="s", num_cores=NC,
        num_subcores=NS)

    @functools.partial(
        pl.kernel,
        out_type=(jax.ShapeDtypeStruct((e, de), jnp.float32),
                  jax.ShapeDtypeStruct((NC, n, de), jnp.float32)),
        mesh=mesh,
        scratch_types=[
            pltpu.VMEM((chunk,), jnp.int32),
            pltpu.VMEM((chunk,), jnp.int32),
            pltpu.VMEM((chunk, de), jnp.float32),
            pltpu.VMEM((chunk, de), jnp.float32),
            pltpu.VMEM((chunk, de), jnp.float32),
            pltpu.VMEM((nrows, de), jnp.float32),
            pltpu.VMEM_SHARED((n, de), jnp.float32),
            pltpu.SemaphoreType.DMA,
            pltpu.SemaphoreType.DMA,
        ],
        compiler_params=pltpu.CompilerParams(use_tc_tiling_on_sc=False),
    )
    def k(p_hbm, q_hbm, ep_hbm, src_hbm, dst_hbm, eout_hbm, agg_hbm,
          src_v, dst_v, pg_v, qg_v, ep_v, zb_v, acc_sh, sem1, sem2):
        cid = lax.axis_index("c")
        sid = lax.axis_index("s")
        wid = sid * NC + cid

        rr = pl.ds(sid * nrows, nrows)

        @pl.loop(0, nrows)
        def _(i):
            zb_v[i] = jnp.zeros((de,), jnp.float32)

        pltpu.sync_copy(zb_v, acc_sh.at[rr])
        plsc.subcore_barrier()

        base0 = wid * ew

        @pl.loop(0, ew // chunk)
        def _(kk):
            base = base0 + kk * chunk
            pltpu.sync_copy(src_hbm.at[pl.ds(base, chunk)], src_v)
            pltpu.sync_copy(dst_hbm.at[pl.ds(base, chunk)], dst_v)
            pltpu.sync_copy(ep_hbm.at[pl.ds(base, chunk)], ep_v)
            cp1 = pltpu.async_copy(p_hbm.at[src_v], pg_v, sem1)
            cp2 = pltpu.async_copy(q_hbm.at[dst_v], qg_v, sem2)
            cp1.wait()
            cp2.wait()

            @pl.loop(0, chunk)
            def _(i):
                ep_v[i] = jnp.maximum(pg_v[i] + qg_v[i] + ep_v[i], 0.0)

            pltpu.sync_copy(ep_v, eout_hbm.at[pl.ds(base, chunk)])
            pltpu.sync_copy(ep_v, acc_sh.at[dst_v], add=True)

        plsc.subcore_barrier()
        pltpu.sync_copy(acc_sh.at[pl.ds(sid * nrows, nrows)], zb_v)
        pltpu.sync_copy(zb_v, agg_hbm.at[cid, pl.ds(sid * nrows, nrows)])

    return k


def _make_sc_spmm(n: int, e: int, d: int, chunk: int):
    ew = e // NW
    nrows = n // NS
    zrows = nrows // 10
    mesh = plsc.VectorSubcoreMesh(
        core_axis_name="c", subcore_axis_name="s", num_cores=NC,
        num_subcores=NS)

    @functools.partial(
        pl.kernel,
        out_type=jax.ShapeDtypeStruct((NC, n, d), jnp.float32),
        mesh=mesh,
        scratch_types=[
            pltpu.VMEM((chunk,), jnp.int32),
            pltpu.VMEM((chunk,), jnp.int32),
            pltpu.VMEM((chunk,), jnp.float32),
            pltpu.VMEM((chunk, d), jnp.float32),
            pltpu.VMEM((zrows, d), jnp.float32),
            pltpu.VMEM_SHARED((n, d), jnp.float32),
            pltpu.SemaphoreType.DMA,
        ],
        compiler_params=pltpu.CompilerParams(use_tc_tiling_on_sc=False,
                                             needs_layout_passes=False),
    )
    def k(hx_hbm, cols_hbm, rows_hbm, vals_hbm, out_hbm,
          cols_v, rows_v, vals_v, g_v, zb_v, acc_sh, sem):
        cid = lax.axis_index("c")
        sid = lax.axis_index("s")
        wid = sid * NC + cid

        @pl.loop(0, zrows)
        def _(i):
            for j in range(d // 16):
                zb_v[i, pl.ds(j * 16, 16)] = jnp.zeros((16,), jnp.float32)

        for kz in range(nrows // zrows):
            pltpu.sync_copy(
                zb_v, acc_sh.at[pl.ds(sid * nrows + kz * zrows, zrows)])
        plsc.subcore_barrier()

        base0 = wid * ew

        @pl.loop(0, ew // chunk)
        def _(kk):
            base = base0 + kk * chunk
            pltpu.sync_copy(cols_hbm.at[pl.ds(base, chunk)], cols_v)
            pltpu.sync_copy(rows_hbm.at[pl.ds(base, chunk)], rows_v)
            pltpu.sync_copy(vals_hbm.at[pl.ds(base, chunk)], vals_v)
            pltpu.async_copy(hx_hbm.at[cols_v], g_v, sem).wait()

            @pl.loop(0, chunk)
            def _(i):
                s = plsc.load_gather(vals_v, [jnp.full((16,), i, jnp.int32)])
                for j in range(d // 16):
                    g_v[i, pl.ds(j * 16, 16)] = g_v[i, pl.ds(j * 16, 16)] * s

            pltpu.sync_copy(g_v, acc_sh.at[rows_v], add=True)

        plsc.subcore_barrier()
        for kz in range(nrows // zrows):
            pltpu.sync_copy(
                acc_sh.at[pl.ds(sid * nrows + kz * zrows, zrows)], zb_v)
            pltpu.sync_copy(
                zb_v, out_hbm.at[cid, pl.ds(sid * nrows + kz * zrows, zrows)])

    return k


def _tc_pre_node(x_ref, hx_ref, wpq_ref, wnx_ref, bn_ref,
                 p_ref, q_ref, s1_ref, *, d, de):
    x = x_ref[...]
    h = hx_ref[...]
    pq = (jnp.dot(x, wpq_ref[0:d], preferred_element_type=jnp.float32)
          + jnp.dot(h, wpq_ref[d:2 * d], preferred_element_type=jnp.float32))
    p_ref[...] = pq[:, 0:de]
    q_ref[...] = pq[:, de:2 * de]
    s1_ref[...] = (jnp.dot(x, wnx_ref[0:d], preferred_element_type=jnp.float32)
                   + jnp.dot(h, wnx_ref[d:2 * d],
                             preferred_element_type=jnp.float32)
                   + bn_ref[...])


def _tc_edge_pre(ea_ref, he_ref, wee_ref, be_ref, ep_ref, *, de):
    ep_ref[...] = (
        jnp.dot(ea_ref[...], wee_ref[0:de], preferred_element_type=jnp.float32)
        + jnp.dot(he_ref[...], wee_ref[de:2 * de],
                  preferred_element_type=jnp.float32)
        + be_ref[...])


def _tc_post_node(s1_ref, a0_ref, a1_ref, hx_ref, sp0_ref, sp1_ref, wna_ref,
                  coeff_ref, xo_ref, td_ref, sp_ref):
    agg = a0_ref[...] + a1_ref[...]
    xo = s1_ref[...] + jnp.dot(agg, wna_ref[...],
                               preferred_element_type=jnp.float32)
    xo_ref[...] = xo
    td_ref[...] = xo - hx_ref[...]
    sp_ref[...] = (-coeff_ref[0, 0]) * (sp0_ref[...] + sp1_ref[...])


def kernel(x_seq, edge_attr_seq, h_x, h_e, lap_vals, We, be, Wn, bn, coeff,
           edge_index, lap_rows, lap_cols):
    t_steps, n, d = x_seq.shape
    e, de = edge_attr_seq.shape[1], edge_attr_seq.shape[2]

    src = edge_index[0]
    dst = edge_index[1]
    wpq = jnp.concatenate([We[0:2 * d], We[2 * d:4 * d]], axis=1)
    wee = We[4 * d:]
    wnx = Wn[0:2 * d]
    wna = Wn[2 * d:]
    be2 = be.reshape(1, de)
    bn2 = bn.reshape(1, d)
    coeff2 = jnp.reshape(coeff, (1, 1))

    bn_blk = 2000
    be_blk = 16000

    pre_node = pl.pallas_call(
        functools.partial(_tc_pre_node, d=d, de=de),
        grid=(n // bn_blk,),
        in_specs=[
            pl.BlockSpec((bn_blk, d), lambda i: (i, 0)),
            pl.BlockSpec((bn_blk, d), lambda i: (i, 0)),
            pl.BlockSpec((2 * d, 2 * de), lambda i: (0, 0)),
            pl.BlockSpec((2 * d, d), lambda i: (0, 0)),
            pl.BlockSpec((1, d), lambda i: (0, 0)),
        ],
        out_specs=[
            pl.BlockSpec((bn_blk, de), lambda i: (i, 0)),
            pl.BlockSpec((bn_blk, de), lambda i: (i, 0)),
            pl.BlockSpec((bn_blk, d), lambda i: (i, 0)),
        ],
        out_shape=[
            jax.ShapeDtypeStruct((n, de), jnp.float32),
            jax.ShapeDtypeStruct((n, de), jnp.float32),
            jax.ShapeDtypeStruct((n, d), jnp.float32),
        ],
    )

    edge_pre = pl.pallas_call(
        functools.partial(_tc_edge_pre, de=de),
        grid=(e // be_blk,),
        in_specs=[
            pl.BlockSpec((be_blk, de), lambda i: (i, 0)),
            pl.BlockSpec((be_blk, de), lambda i: (i, 0)),
            pl.BlockSpec((2 * de, de), lambda i: (0, 0)),
            pl.BlockSpec((1, de), lambda i: (0, 0)),
        ],
        out_specs=pl.BlockSpec((be_blk, de), lambda i: (i, 0)),
        out_shape=jax.ShapeDtypeStruct((e, de), jnp.float32),
    )

    post_node = pl.pallas_call(
        _tc_post_node,
        grid=(n // bn_blk,),
        in_specs=[
            pl.BlockSpec((bn_blk, d), lambda i: (i, 0)),
            pl.BlockSpec((bn_blk, de), lambda i: (i, 0)),
            pl.BlockSpec((bn_blk, de), lambda i: (i, 0)),
            pl.BlockSpec((bn_blk, d), lambda i: (i, 0)),
            pl.BlockSpec((bn_blk, d), lambda i: (i, 0)),
            pl.BlockSpec((bn_blk, d), lambda i: (i, 0)),
            pl.BlockSpec((de, d), lambda i: (0, 0)),
            pl.BlockSpec(memory_space=pltpu.SMEM),
        ],
        out_specs=[
            pl.BlockSpec((bn_blk, d), lambda i: (i, 0)),
            pl.BlockSpec((bn_blk, d), lambda i: (i, 0)),
            pl.BlockSpec((bn_blk, d), lambda i: (i, 0)),
        ],
        out_shape=[
            jax.ShapeDtypeStruct((n, d), jnp.float32),
            jax.ShapeDtypeStruct((n, d), jnp.float32),
            jax.ShapeDtypeStruct((n, d), jnp.float32),
        ],
    )

    n_pad = ((n + 639) // 640) * 640
    sc_edge = _make_sc_edge(n_pad, e, de, chunk=1000)
    sc_spmm = _make_sc_spmm(n_pad, e, d, chunk=200)

    hx, he = h_x, h_e
    out_x, out_e, tds, sps = [], [], [], []
    for t in range(t_steps):
        p, q, s1 = pre_node(x_seq[t], hx, wpq, wnx, bn2)
        p = jnp.pad(p, ((0, n_pad - n), (0, 0)))
        q = jnp.pad(q, ((0, n_pad - n), (0, 0)))
        epart = edge_pre(edge_attr_seq[t], he, wee, be2)
        e_out, agg2 = sc_edge(p, q, epart, src, dst)
        sp2 = sc_spmm(hx, lap_cols, lap_rows, lap_vals)
        x_out, td, sp = post_node(s1, agg2[0, :n], agg2[1, :n], hx,
                                  sp2[0, :n], sp2[1, :n], wna, coeff2)
        hx, he = x_out, e_out
        out_x.append(x_out)
        out_e.append(e_out)
        tds.append(td)
        sps.append(sp)

    return (jnp.stack(out_x), jnp.stack(out_e), jnp.stack(tds),
            jnp.stack(sps))


# speedup vs baseline: 1.1813x; 1.1813x over previous
import functools

import jax
import jax.numpy as jnp
from jax import lax
from jax.experimental import pallas as pl
from jax.experimental.pallas import tpu as pltpu
from jax.experimental.pallas import tpu_sc as plsc

NC = 2
NS = 16
NW = NC * NS


def _make_sc_edge(n: int, e: int, de: int, chunk: int):
    ew = e // NW
    nrows = n // NS
    mesh = plsc.VectorSubcoreMesh(
        core_axis_name="c", subcore_axis_name="s", num_cores=NC,
        num_subcores=NS)

    kk = ew // chunk

    @functools.partial(
        pl.kernel,
        out_type=(jax.ShapeDtypeStruct((e, de), jnp.float32),
                  jax.ShapeDtypeStruct((NC, n, de), jnp.float32)),
        mesh=mesh,
        scratch_types=[
            [pltpu.VMEM((chunk,), jnp.int32) for _ in range(3)],
            [pltpu.VMEM((chunk,), jnp.int32) for _ in range(3)],
            [pltpu.VMEM((chunk, de), jnp.float32) for _ in range(2)],
            [pltpu.VMEM((chunk, de), jnp.float32) for _ in range(2)],
            [pltpu.VMEM((chunk, de), jnp.float32) for _ in range(2)],
            pltpu.VMEM((nrows, de), jnp.float32),
            pltpu.VMEM_SHARED((n, de), jnp.float32),
            [pltpu.SemaphoreType.DMA for _ in range(3)],
            [pltpu.SemaphoreType.DMA for _ in range(2)],
            [pltpu.SemaphoreType.DMA for _ in range(2)],
            [pltpu.SemaphoreType.DMA for _ in range(2)],
        ],
        compiler_params=pltpu.CompilerParams(use_tc_tiling_on_sc=False),
    )
    def k(p_hbm, q_hbm, ep_hbm, src_hbm, dst_hbm, eout_hbm, agg_hbm,
          src_v, dst_v, pg_v, qg_v, ep_v, zb_v, acc_sh,
          sem_i, sem_p, sem_q, sem_e):
        cid = lax.axis_index("c")
        sid = lax.axis_index("s")
        wid = sid * NC + cid

        rr = pl.ds(sid * nrows, nrows)

        @pl.loop(0, nrows)
        def _(i):
            zb_v[i] = jnp.zeros((de,), jnp.float32)

        pltpu.sync_copy(zb_v, acc_sh.at[rr])
        plsc.subcore_barrier()

        base0 = wid * ew

        def start_idx(c):
            s3 = c % 3
            a = pltpu.async_copy(src_hbm.at[pl.ds(base0 + c * chunk, chunk)],
                                 src_v[s3], sem_i[s3])
            b = pltpu.async_copy(dst_hbm.at[pl.ds(base0 + c * chunk, chunk)],
                                 dst_v[s3], sem_i[s3])
            return (a, b)

        def start_gathers(c):
            s3, s2 = c % 3, c % 2
            g1 = pltpu.async_copy(p_hbm.at[src_v[s3]], pg_v[s2], sem_p[s2])
            g2 = pltpu.async_copy(q_hbm.at[dst_v[s3]], qg_v[s2], sem_q[s2])
            g3 = pltpu.async_copy(
                ep_hbm.at[pl.ds(base0 + c * chunk, chunk)], ep_v[s2],
                sem_e[s2])
            return (g1, g2, g3)

        idx_d = {}
        g_d = {}
        for c in range(kk):
            s3, s2 = c % 3, c % 2
            if c == 0:
                for d in start_idx(0):
                    d.wait()
                g_d[0] = start_gathers(0)
                idx_d[1] = start_idx(1)
                idx_d[2] = start_idx(2)
            if c + 1 < kk:
                for d in idx_d.pop(c + 1):
                    d.wait()
                g_d[c + 1] = start_gathers(c + 1)
            for d in g_d.pop(c):
                d.wait()

            @pl.loop(0, chunk)
            def _(i):
                ep_v[s2][i] = jnp.maximum(
                    pg_v[s2][i] + qg_v[s2][i] + ep_v[s2][i], 0.0)

            pltpu.sync_copy(ep_v[s2],
                            eout_hbm.at[pl.ds(base0 + c * chunk, chunk)])
            pltpu.sync_copy(ep_v[s2], acc_sh.at[dst_v[s3]], add=True)
            if c + 3 < kk:
                idx_d[c + 3] = start_idx(c + 3)

        plsc.subcore_barrier()
        pltpu.sync_copy(acc_sh.at[rr], zb_v)
        pltpu.sync_copy(zb_v, agg_hbm.at[cid, rr])

    return k


def _make_sc_spmm(n: int, e: int, d: int, chunk: int):
    ew = e // NW
    nrows = n // NS
    kk = ew // chunk
    nz = nrows // chunk
    mesh = plsc.VectorSubcoreMesh(
        core_axis_name="c", subcore_axis_name="s", num_cores=NC,
        num_subcores=NS)

    @functools.partial(
        pl.kernel,
        out_type=jax.ShapeDtypeStruct((NC, n, d), jnp.float32),
        mesh=mesh,
        scratch_types=[
            [pltpu.VMEM((chunk,), jnp.int32) for _ in range(3)],
            [pltpu.VMEM((chunk,), jnp.int32) for _ in range(3)],
            [pltpu.VMEM((chunk,), jnp.float32) for _ in range(3)],
            [pltpu.VMEM((chunk, d), jnp.float32) for _ in range(2)],
            pltpu.VMEM_SHARED((n, d), jnp.float32),
            [pltpu.SemaphoreType.DMA for _ in range(3)],
            [pltpu.SemaphoreType.DMA for _ in range(2)],
        ],
        compiler_params=pltpu.CompilerParams(use_tc_tiling_on_sc=False,
                                             needs_layout_passes=False),
    )
    def k(hx_hbm, cols_hbm, rows_hbm, vals_hbm, out_hbm,
          cols_v, rows_v, vals_v, g_v, acc_sh, sem_i, sem_g):
        cid = lax.axis_index("c")
        sid = lax.axis_index("s")
        wid = sid * NC + cid

        @pl.loop(0, chunk)
        def _(i):
            for j in range(d // 16):
                g_v[0][i, pl.ds(j * 16, 16)] = jnp.zeros((16,), jnp.float32)

        for kz in range(nz):
            pltpu.sync_copy(
                g_v[0], acc_sh.at[pl.ds(sid * nrows + kz * chunk, chunk)])
        plsc.subcore_barrier()

        base0 = wid * ew

        def start_idx(c):
            s3 = c % 3
            base = pl.ds(base0 + c * chunk, chunk)
            return (pltpu.async_copy(cols_hbm.at[base], cols_v[s3],
                                     sem_i[s3]),
                    pltpu.async_copy(rows_hbm.at[base], rows_v[s3],
                                     sem_i[s3]),
                    pltpu.async_copy(vals_hbm.at[base], vals_v[s3],
                                     sem_i[s3]))

        def start_gather(c):
            s3, s2 = c % 3, c % 2
            return (pltpu.async_copy(hx_hbm.at[cols_v[s3]], g_v[s2],
                                     sem_g[s2]),)

        idx_d = {}
        g_d = {}
        for c in range(kk):
            s3, s2 = c % 3, c % 2
            if c == 0:
                for dd in start_idx(0):
                    dd.wait()
                g_d[0] = start_gather(0)
                idx_d[1] = start_idx(1)
                idx_d[2] = start_idx(2)
            if c + 1 < kk:
                for dd in idx_d.pop(c + 1):
                    dd.wait()
                g_d[c + 1] = start_gather(c + 1)
            for dd in g_d.pop(c):
                dd.wait()

            @pl.loop(0, chunk)
            def _(i):
                s = plsc.load_gather(vals_v[s3],
                                     [jnp.full((16,), i, jnp.int32)])
                for j in range(d // 16):
                    g_v[s2][i, pl.ds(j * 16, 16)] = (
                        g_v[s2][i, pl.ds(j * 16, 16)] * s)

            pltpu.sync_copy(g_v[s2], acc_sh.at[rows_v[s3]], add=True)
            if c + 3 < kk:
                idx_d[c + 3] = start_idx(c + 3)

        plsc.subcore_barrier()
        for kz in range(nz):
            sl = pl.ds(sid * nrows + kz * chunk, chunk)
            pltpu.sync_copy(acc_sh.at[sl], g_v[0])
            pltpu.sync_copy(g_v[0], out_hbm.at[cid, sl])

    return k


def _tc_pre_node(x_ref, hx_ref, wpq_ref, wnx_ref, bn_ref,
                 p_ref, q_ref, s1_ref, *, d, de):
    x = x_ref[...]
    h = hx_ref[...]
    pq = (jnp.dot(x, wpq_ref[0:d], preferred_element_type=jnp.float32)
          + jnp.dot(h, wpq_ref[d:2 * d], preferred_element_type=jnp.float32))
    p_ref[...] = pq[:, 0:de]
    q_ref[...] = pq[:, de:2 * de]
    s1_ref[...] = (jnp.dot(x, wnx_ref[0:d], preferred_element_type=jnp.float32)
                   + jnp.dot(h, wnx_ref[d:2 * d],
                             preferred_element_type=jnp.float32)
                   + bn_ref[...])


def _tc_edge_pre(ea_ref, he_ref, wee_ref, be_ref, ep_ref, *, de):
    ep_ref[...] = (
        jnp.dot(ea_ref[...], wee_ref[0:de], preferred_element_type=jnp.float32)
        + jnp.dot(he_ref[...], wee_ref[de:2 * de],
                  preferred_element_type=jnp.float32)
        + be_ref[...])


def _tc_post_node(s1_ref, a0_ref, a1_ref, hx_ref, sp0_ref, sp1_ref, wna_ref,
                  coeff_ref, xo_ref, td_ref, sp_ref):
    agg = a0_ref[...] + a1_ref[...]
    xo = s1_ref[...] + jnp.dot(agg, wna_ref[...],
                               preferred_element_type=jnp.float32)
    xo_ref[...] = xo
    td_ref[...] = xo - hx_ref[...]
    sp_ref[...] = (-coeff_ref[0, 0]) * (sp0_ref[...] + sp1_ref[...])


def kernel(x_seq, edge_attr_seq, h_x, h_e, lap_vals, We, be, Wn, bn, coeff,
           edge_index, lap_rows, lap_cols):
    t_steps, n, d = x_seq.shape
    e, de = edge_attr_seq.shape[1], edge_attr_seq.shape[2]

    src = edge_index[0]
    dst = edge_index[1]
    wpq = jnp.concatenate([We[0:2 * d], We[2 * d:4 * d]], axis=1)
    wee = We[4 * d:]
    wnx = Wn[0:2 * d]
    wna = Wn[2 * d:]
    be2 = be.reshape(1, de)
    bn2 = bn.reshape(1, d)
    coeff2 = jnp.reshape(coeff, (1, 1))

    bn_blk = 2000
    be_blk = 16000

    pre_node = pl.pallas_call(
        functools.partial(_tc_pre_node, d=d, de=de),
        grid=(n // bn_blk,),
        in_specs=[
            pl.BlockSpec((bn_blk, d), lambda i: (i, 0)),
            pl.BlockSpec((bn_blk, d), lambda i: (i, 0)),
            pl.BlockSpec((2 * d, 2 * de), lambda i: (0, 0)),
            pl.BlockSpec((2 * d, d), lambda i: (0, 0)),
            pl.BlockSpec((1, d), lambda i: (0, 0)),
        ],
        out_specs=[
            pl.BlockSpec((bn_blk, de), lambda i: (i, 0)),
            pl.BlockSpec((bn_blk, de), lambda i: (i, 0)),
            pl.BlockSpec((bn_blk, d), lambda i: (i, 0)),
        ],
        out_shape=[
            jax.ShapeDtypeStruct((n, de), jnp.float32),
            jax.ShapeDtypeStruct((n, de), jnp.float32),
            jax.ShapeDtypeStruct((n, d), jnp.float32),
        ],
    )

    edge_pre = pl.pallas_call(
        functools.partial(_tc_edge_pre, de=de),
        grid=(e // be_blk,),
        in_specs=[
            pl.BlockSpec((be_blk, de), lambda i: (i, 0)),
            pl.BlockSpec((be_blk, de), lambda i: (i, 0)),
            pl.BlockSpec((2 * de, de), lambda i: (0, 0)),
            pl.BlockSpec((1, de), lambda i: (0, 0)),
        ],
        out_specs=pl.BlockSpec((be_blk, de), lambda i: (i, 0)),
        out_shape=jax.ShapeDtypeStruct((e, de), jnp.float32),
    )

    post_node = pl.pallas_call(
        _tc_post_node,
        grid=(n // bn_blk,),
        in_specs=[
            pl.BlockSpec((bn_blk, d), lambda i: (i, 0)),
            pl.BlockSpec((bn_blk, de), lambda i: (i, 0)),
            pl.BlockSpec((bn_blk, de), lambda i: (i, 0)),
            pl.BlockSpec((bn_blk, d), lambda i: (i, 0)),
            pl.BlockSpec((bn_blk, d), lambda i: (i, 0)),
            pl.BlockSpec((bn_blk, d), lambda i: (i, 0)),
            pl.BlockSpec((de, d), lambda i: (0, 0)),
            pl.BlockSpec(memory_space=pltpu.SMEM),
        ],
        out_specs=[
            pl.BlockSpec((bn_blk, d), lambda i: (i, 0)),
            pl.BlockSpec((bn_blk, d), lambda i: (i, 0)),
            pl.BlockSpec((bn_blk, d), lambda i: (i, 0)),
        ],
        out_shape=[
            jax.ShapeDtypeStruct((n, d), jnp.float32),
            jax.ShapeDtypeStruct((n, d), jnp.float32),
            jax.ShapeDtypeStruct((n, d), jnp.float32),
        ],
    )

    n_pad = ((n + 639) // 640) * 640
    sc_edge = _make_sc_edge(n_pad, e, de, chunk=1000)
    sc_spmm = _make_sc_spmm(n_pad, e, d, chunk=80)

    hx, he = h_x, h_e
    out_x, out_e, tds, sps = [], [], [], []
    for t in range(t_steps):
        p, q, s1 = pre_node(x_seq[t], hx, wpq, wnx, bn2)
        p = jnp.pad(p, ((0, n_pad - n), (0, 0)))
        q = jnp.pad(q, ((0, n_pad - n), (0, 0)))
        epart = edge_pre(edge_attr_seq[t], he, wee, be2)
        e_out, agg2 = sc_edge(p, q, epart, src, dst)
        sp2 = sc_spmm(hx, lap_cols, lap_rows, lap_vals)
        x_out, td, sp = post_node(s1, agg2[0, :n], agg2[1, :n], hx,
                                  sp2[0, :n], sp2[1, :n], wna, coeff2)
        hx, he = x_out, e_out
        out_x.append(x_out)
        out_e.append(e_out)
        tds.append(td)
        sps.append(sp)

    return (jnp.stack(out_x), jnp.stack(out_e), jnp.stack(tds),
            jnp.stack(sps))


# speedup vs baseline: 1.6447x; 1.3923x over previous
import functools

import jax
import jax.numpy as jnp
from jax import lax
from jax.experimental import pallas as pl
from jax.experimental.pallas import tpu as pltpu
from jax.experimental.pallas import tpu_sc as plsc

NC = 2
NS = 16
NW = NC * NS


def _make_sc_edge(n: int, e: int, de: int, chunk: int):
    ew = e // NW
    nrows = n // NS
    mesh = plsc.VectorSubcoreMesh(
        core_axis_name="c", subcore_axis_name="s", num_cores=NC,
        num_subcores=NS)

    kk = ew // chunk

    @functools.partial(
        pl.kernel,
        out_type=(jax.ShapeDtypeStruct((e, de), jnp.float32),
                  jax.ShapeDtypeStruct((NC, n, de), jnp.float32)),
        mesh=mesh,
        scratch_types=[
            [pltpu.VMEM((chunk,), jnp.int32) for _ in range(3)],
            [pltpu.VMEM((chunk,), jnp.int32) for _ in range(3)],
            [pltpu.VMEM((chunk, de), jnp.float32) for _ in range(2)],
            [pltpu.VMEM((chunk, de), jnp.float32) for _ in range(2)],
            [pltpu.VMEM((chunk // 8, 8 * de), jnp.float32)
             for _ in range(2)],
            pltpu.VMEM((nrows, de), jnp.float32),
            pltpu.VMEM_SHARED((n, de), jnp.float32),
            [pltpu.SemaphoreType.DMA for _ in range(3)],
            [pltpu.SemaphoreType.DMA for _ in range(2)],
            [pltpu.SemaphoreType.DMA for _ in range(2)],
            [pltpu.SemaphoreType.DMA for _ in range(2)],
        ],
        compiler_params=pltpu.CompilerParams(use_tc_tiling_on_sc=False),
    )
    def k(p_hbm, q_hbm, ep_hbm, src_hbm, dst_hbm, eout_hbm, agg_hbm,
          src_v, dst_v, pg_v, qg_v, ep_v, zb_v, acc_sh,
          sem_i, sem_p, sem_q, sem_e):
        cid = lax.axis_index("c")
        sid = lax.axis_index("s")
        wid = sid * NC + cid

        rr = pl.ds(sid * nrows, nrows)

        @pl.loop(0, nrows)
        def _(i):
            zb_v[i] = jnp.zeros((de,), jnp.float32)

        pltpu.sync_copy(zb_v, acc_sh.at[rr])
        plsc.subcore_barrier()

        base0 = wid * ew

        def start_idx(c):
            s3 = c % 3
            a = pltpu.async_copy(src_hbm.at[pl.ds(base0 + c * chunk, chunk)],
                                 src_v[s3], sem_i[s3])
            b = pltpu.async_copy(dst_hbm.at[pl.ds(base0 + c * chunk, chunk)],
                                 dst_v[s3], sem_i[s3])
            return (a, b)

        def start_gathers(c):
            s3, s2 = c % 3, c % 2
            g1 = pltpu.async_copy(p_hbm.at[src_v[s3]], pg_v[s2], sem_p[s2])
            g2 = pltpu.async_copy(q_hbm.at[dst_v[s3]], qg_v[s2], sem_q[s2])
            g3 = pltpu.async_copy(
                ep_hbm.at[pl.ds((base0 + c * chunk) // 8, chunk // 8)],
                ep_v[s2], sem_e[s2])
            return (g1, g2, g3)

        idx_d = {}
        g_d = {}
        for c in range(kk):
            s3, s2 = c % 3, c % 2
            if c == 0:
                for d in start_idx(0):
                    d.wait()
                g_d[0] = start_gathers(0)
                idx_d[1] = start_idx(1)
                idx_d[2] = start_idx(2)
            if c + 1 < kk:
                for d in idx_d.pop(c + 1):
                    d.wait()
                g_d[c + 1] = start_gathers(c + 1)
            for d in g_d.pop(c):
                d.wait()

            @pl.loop(0, chunk // 8)
            def _(i):
                for j in range(8):
                    pg_v[s2][8 * i + j] = jnp.maximum(
                        pg_v[s2][8 * i + j] + qg_v[s2][8 * i + j]
                        + ep_v[s2][i, pl.ds(de * j, de)], 0.0)

            pltpu.sync_copy(pg_v[s2],
                            eout_hbm.at[pl.ds(base0 + c * chunk, chunk)])
            pltpu.sync_copy(pg_v[s2], acc_sh.at[dst_v[s3]], add=True)
            if c + 3 < kk:
                idx_d[c + 3] = start_idx(c + 3)

        plsc.subcore_barrier()
        pltpu.sync_copy(acc_sh.at[rr], zb_v)
        pltpu.sync_copy(zb_v, agg_hbm.at[cid, rr])

    return k


def _make_sc_spmm(n: int, e: int, d: int, chunk: int):
    ew = e // NW
    nrows = n // NS
    kk = ew // chunk
    nz = nrows // chunk
    mesh = plsc.VectorSubcoreMesh(
        core_axis_name="c", subcore_axis_name="s", num_cores=NC,
        num_subcores=NS)

    @functools.partial(
        pl.kernel,
        out_type=jax.ShapeDtypeStruct((NC, n, d), jnp.float32),
        mesh=mesh,
        scratch_types=[
            [pltpu.VMEM((chunk,), jnp.int32) for _ in range(3)],
            [pltpu.VMEM((chunk,), jnp.int32) for _ in range(3)],
            [pltpu.VMEM((chunk,), jnp.float32) for _ in range(3)],
            [pltpu.VMEM((chunk, d), jnp.float32) for _ in range(2)],
            pltpu.VMEM_SHARED((n, d), jnp.float32),
            [pltpu.SemaphoreType.DMA for _ in range(3)],
            [pltpu.SemaphoreType.DMA for _ in range(2)],
        ],
        compiler_params=pltpu.CompilerParams(use_tc_tiling_on_sc=False,
                                             needs_layout_passes=False),
    )
    def k(hx_hbm, cols_hbm, rows_hbm, vals_hbm, out_hbm,
          cols_v, rows_v, vals_v, g_v, acc_sh, sem_i, sem_g):
        cid = lax.axis_index("c")
        sid = lax.axis_index("s")
        wid = sid * NC + cid

        @pl.loop(0, chunk)
        def _(i):
            for j in range(d // 16):
                g_v[0][i, pl.ds(j * 16, 16)] = jnp.zeros((16,), jnp.float32)

        for kz in range(nz):
            pltpu.sync_copy(
                g_v[0], acc_sh.at[pl.ds(sid * nrows + kz * chunk, chunk)])
        plsc.subcore_barrier()

        base0 = wid * ew

        def start_idx(c):
            s3 = c % 3
            base = pl.ds(base0 + c * chunk, chunk)
            return (pltpu.async_copy(cols_hbm.at[base], cols_v[s3],
                                     sem_i[s3]),
                    pltpu.async_copy(rows_hbm.at[base], rows_v[s3],
                                     sem_i[s3]),
                    pltpu.async_copy(vals_hbm.at[base], vals_v[s3],
                                     sem_i[s3]))

        def start_gather(c):
            s3, s2 = c % 3, c % 2
            return (pltpu.async_copy(hx_hbm.at[cols_v[s3]], g_v[s2],
                                     sem_g[s2]),)

        idx_d = {}
        g_d = {}
        for c in range(kk):
            s3, s2 = c % 3, c % 2
            if c == 0:
                for dd in start_idx(0):
                    dd.wait()
                g_d[0] = start_gather(0)
                idx_d[1] = start_idx(1)
                idx_d[2] = start_idx(2)
            if c + 1 < kk:
                for dd in idx_d.pop(c + 1):
                    dd.wait()
                g_d[c + 1] = start_gather(c + 1)
            for dd in g_d.pop(c):
                dd.wait()

            @pl.loop(0, chunk)
            def _(i):
                s = plsc.load_gather(vals_v[s3],
                                     [jnp.full((16,), i, jnp.int32)])
                for j in range(d // 16):
                    g_v[s2][i, pl.ds(j * 16, 16)] = (
                        g_v[s2][i, pl.ds(j * 16, 16)] * s)

            pltpu.sync_copy(g_v[s2], acc_sh.at[rows_v[s3]], add=True)
            if c + 3 < kk:
                idx_d[c + 3] = start_idx(c + 3)

        plsc.subcore_barrier()
        for kz in range(nz):
            sl = pl.ds(sid * nrows + kz * chunk, chunk)
            pltpu.sync_copy(acc_sh.at[sl], g_v[0])
            pltpu.sync_copy(g_v[0], out_hbm.at[cid, sl])

    return k


def _tc_pre_node(x_ref, hx_ref, wpq_ref, wnx_ref, bn_ref,
                 p_ref, q_ref, s1_ref, *, d, de):
    x = x_ref[...]
    h = hx_ref[...]
    pq = (jnp.dot(x, wpq_ref[0:d], preferred_element_type=jnp.float32)
          + jnp.dot(h, wpq_ref[d:2 * d], preferred_element_type=jnp.float32))
    p_ref[...] = pq[:, 0:de]
    q_ref[...] = pq[:, de:2 * de]
    s1_ref[...] = (jnp.dot(x, wnx_ref[0:d], preferred_element_type=jnp.float32)
                   + jnp.dot(h, wnx_ref[d:2 * d],
                             preferred_element_type=jnp.float32)
                   + bn_ref[...])


def _tc_edge_pre(ea_ref, he_ref, w1_ref, w2_ref, be_ref, ep_ref):
    ep_ref[...] = (
        jnp.dot(ea_ref[...], w1_ref[...], preferred_element_type=jnp.float32)
        + jnp.dot(he_ref[...], w2_ref[...],
                  preferred_element_type=jnp.float32)
        + be_ref[...])


def _tc_post_node(s1_ref, a0_ref, a1_ref, hx_ref, sp0_ref, sp1_ref, wna_ref,
                  coeff_ref, xo_ref, td_ref, sp_ref):
    agg = a0_ref[...] + a1_ref[...]
    xo = s1_ref[...] + jnp.dot(agg, wna_ref[...],
                               preferred_element_type=jnp.float32)
    xo_ref[...] = xo
    td_ref[...] = xo - hx_ref[...]
    sp_ref[...] = (-coeff_ref[0, 0]) * (sp0_ref[...] + sp1_ref[...])


def kernel(x_seq, edge_attr_seq, h_x, h_e, lap_vals, We, be, Wn, bn, coeff,
           edge_index, lap_rows, lap_cols):
    t_steps, n, d = x_seq.shape
    e, de = edge_attr_seq.shape[1], edge_attr_seq.shape[2]

    src = edge_index[0]
    dst = edge_index[1]
    er = e // 8
    wpq = jnp.concatenate([We[0:2 * d], We[2 * d:4 * d]], axis=1)
    eye8 = jnp.eye(8, dtype=jnp.float32)
    wee1 = jnp.kron(eye8, We[4 * d:4 * d + de])
    wee2 = jnp.kron(eye8, We[4 * d + de:])
    wnx = Wn[0:2 * d]
    wna = Wn[2 * d:]
    be_p = jnp.tile(be, 8).reshape(1, 8 * de)
    bn2 = bn.reshape(1, d)
    coeff2 = jnp.reshape(coeff, (1, 1))
    ea_p = jnp.reshape(edge_attr_seq, (t_steps, er, 8 * de))
    he_p = jnp.reshape(h_e, (er, 8 * de))

    bn_blk = 2000
    be_blk = 16000

    pre_node = pl.pallas_call(
        functools.partial(_tc_pre_node, d=d, de=de),
        grid=(n // bn_blk,),
        in_specs=[
            pl.BlockSpec((bn_blk, d), lambda i: (i, 0)),
            pl.BlockSpec((bn_blk, d), lambda i: (i, 0)),
            pl.BlockSpec((2 * d, 2 * de), lambda i: (0, 0)),
            pl.BlockSpec((2 * d, d), lambda i: (0, 0)),
            pl.BlockSpec((1, d), lambda i: (0, 0)),
        ],
        out_specs=[
            pl.BlockSpec((bn_blk, de), lambda i: (i, 0)),
            pl.BlockSpec((bn_blk, de), lambda i: (i, 0)),
            pl.BlockSpec((bn_blk, d), lambda i: (i, 0)),
        ],
        out_shape=[
            jax.ShapeDtypeStruct((n, de), jnp.float32),
            jax.ShapeDtypeStruct((n, de), jnp.float32),
            jax.ShapeDtypeStruct((n, d), jnp.float32),
        ],
    )

    ber_blk = be_blk // 8
    edge_pre = pl.pallas_call(
        _tc_edge_pre,
        grid=(er // ber_blk,),
        in_specs=[
            pl.BlockSpec((ber_blk, 8 * de), lambda i: (i, 0)),
            pl.BlockSpec((ber_blk, 8 * de), lambda i: (i, 0)),
            pl.BlockSpec((8 * de, 8 * de), lambda i: (0, 0)),
            pl.BlockSpec((8 * de, 8 * de), lambda i: (0, 0)),
            pl.BlockSpec((1, 8 * de), lambda i: (0, 0)),
        ],
        out_specs=pl.BlockSpec((ber_blk, 8 * de), lambda i: (i, 0)),
        out_shape=jax.ShapeDtypeStruct((er, 8 * de), jnp.float32),
    )

    post_node = pl.pallas_call(
        _tc_post_node,
        grid=(n // bn_blk,),
        in_specs=[
            pl.BlockSpec((bn_blk, d), lambda i: (i, 0)),
            pl.BlockSpec((bn_blk, de), lambda i: (i, 0)),
            pl.BlockSpec((bn_blk, de), lambda i: (i, 0)),
            pl.BlockSpec((bn_blk, d), lambda i: (i, 0)),
            pl.BlockSpec((bn_blk, d), lambda i: (i, 0)),
            pl.BlockSpec((bn_blk, d), lambda i: (i, 0)),
            pl.BlockSpec((de, d), lambda i: (0, 0)),
            pl.BlockSpec(memory_space=pltpu.SMEM),
        ],
        out_specs=[
            pl.BlockSpec((bn_blk, d), lambda i: (i, 0)),
            pl.BlockSpec((bn_blk, d), lambda i: (i, 0)),
            pl.BlockSpec((bn_blk, d), lambda i: (i, 0)),
        ],
        out_shape=[
            jax.ShapeDtypeStruct((n, d), jnp.float32),
            jax.ShapeDtypeStruct((n, d), jnp.float32),
            jax.ShapeDtypeStruct((n, d), jnp.float32),
        ],
    )

    n_pad = ((n + 639) // 640) * 640
    sc_edge = _make_sc_edge(n_pad, e, de, chunk=1000)
    sc_spmm = _make_sc_spmm(n_pad, e, d, chunk=80)

    hx, he = h_x, he_p
    out_x, out_e, tds, sps = [], [], [], []
    for t in range(t_steps):
        p, q, s1 = pre_node(x_seq[t], hx, wpq, wnx, bn2)
        p = jnp.pad(p, ((0, n_pad - n), (0, 0)))
        q = jnp.pad(q, ((0, n_pad - n), (0, 0)))
        epart = edge_pre(ea_p[t], he, wee1, wee2, be_p)
        e_out, agg2 = sc_edge(p, q, epart, src, dst)
        sp2 = sc_spmm(hx, lap_cols, lap_rows, lap_vals)
        x_out, td, sp = post_node(s1, agg2[0, :n], agg2[1, :n], hx,
                                  sp2[0, :n], sp2[1, :n], wna, coeff2)
        hx, he = x_out, jnp.reshape(e_out, (er, 8 * de))
        out_x.append(x_out)
        out_e.append(e_out)
        tds.append(td)
        sps.append(sp)

    return (jnp.stack(out_x), jnp.stack(out_e), jnp.stack(tds),
            jnp.stack(sps))


# speedup vs baseline: 1.6679x; 1.0141x over previous
import functools

import jax
import jax.numpy as jnp
from jax import lax
from jax.experimental import pallas as pl
from jax.experimental.pallas import tpu as pltpu
from jax.experimental.pallas import tpu_sc as plsc

NC = 2
NS = 16
NW = NC * NS


def _make_sc_edge(n: int, e: int, de: int, chunk: int):
    ew = e // NW
    nrows = n // NS
    mesh = plsc.VectorSubcoreMesh(
        core_axis_name="c", subcore_axis_name="s", num_cores=NC,
        num_subcores=NS)

    kk = ew // chunk

    @functools.partial(
        pl.kernel,
        out_type=(jax.ShapeDtypeStruct((e, de), jnp.float32),
                  jax.ShapeDtypeStruct((NC, n, de), jnp.float32)),
        mesh=mesh,
        scratch_types=[
            [pltpu.VMEM((chunk,), jnp.int32) for _ in range(4)],
            [pltpu.VMEM((chunk,), jnp.int32) for _ in range(4)],
            [pltpu.VMEM((chunk, de), jnp.float32) for _ in range(2)],
            [pltpu.VMEM((chunk, de), jnp.float32) for _ in range(2)],
            [pltpu.VMEM((chunk // 8, 8 * de), jnp.float32)
             for _ in range(2)],
            pltpu.VMEM((nrows, de), jnp.float32),
            pltpu.VMEM_SHARED((n, de), jnp.float32),
            [pltpu.SemaphoreType.DMA for _ in range(4)],
            [pltpu.SemaphoreType.DMA for _ in range(2)],
            [pltpu.SemaphoreType.DMA for _ in range(2)],
            [pltpu.SemaphoreType.DMA for _ in range(2)],
            [pltpu.SemaphoreType.DMA for _ in range(2)],
            [pltpu.SemaphoreType.DMA for _ in range(2)],
        ],
        compiler_params=pltpu.CompilerParams(use_tc_tiling_on_sc=False),
    )
    def k(p_hbm, q_hbm, ep_hbm, src_hbm, dst_hbm, eout_hbm, agg_hbm,
          src_v, dst_v, pg_v, qg_v, ep_v, zb_v, acc_sh,
          sem_i, sem_p, sem_q, sem_e, sem_w, sem_s):
        cid = lax.axis_index("c")
        sid = lax.axis_index("s")
        wid = sid * NC + cid

        rr = pl.ds(sid * nrows, nrows)

        @pl.loop(0, nrows)
        def _(i):
            zb_v[i] = jnp.zeros((de,), jnp.float32)

        pltpu.sync_copy(zb_v, acc_sh.at[rr])
        plsc.subcore_barrier()

        base0 = wid * ew

        def start_idx(c):
            s4 = c % 4
            a = pltpu.async_copy(src_hbm.at[pl.ds(base0 + c * chunk, chunk)],
                                 src_v[s4], sem_i[s4])
            b = pltpu.async_copy(dst_hbm.at[pl.ds(base0 + c * chunk, chunk)],
                                 dst_v[s4], sem_i[s4])
            return (a, b)

        def start_gathers(c):
            s4, s2 = c % 4, c % 2
            g1 = pltpu.async_copy(p_hbm.at[src_v[s4]], pg_v[s2], sem_p[s2])
            g2 = pltpu.async_copy(q_hbm.at[dst_v[s4]], qg_v[s2], sem_q[s2])
            g3 = pltpu.async_copy(
                ep_hbm.at[pl.ds((base0 + c * chunk) // 8, chunk // 8)],
                ep_v[s2], sem_e[s2])
            return (g1, g2, g3)

        idx_d = {}
        g_d = {}
        w_d = {}
        for c in range(kk):
            s4, s2 = c % 4, c % 2
            if c == 0:
                for d in start_idx(0):
                    d.wait()
                g_d[0] = start_gathers(0)
                for nx in (1, 2, 3):
                    idx_d[nx] = start_idx(nx)
            if c >= 1:
                for d in w_d.pop(c - 1):
                    d.wait()
                if c + 3 < kk:
                    idx_d[c + 3] = start_idx(c + 3)
            if c + 1 < kk:
                for d in idx_d.pop(c + 1):
                    d.wait()
                g_d[c + 1] = start_gathers(c + 1)
            for d in g_d.pop(c):
                d.wait()

            @pl.loop(0, chunk // 8)
            def _(i):
                for j in range(8):
                    pg_v[s2][8 * i + j] = jnp.maximum(
                        pg_v[s2][8 * i + j] + qg_v[s2][8 * i + j]
                        + ep_v[s2][i, pl.ds(de * j, de)], 0.0)

            w_d[c] = (
                pltpu.async_copy(
                    pg_v[s2], eout_hbm.at[pl.ds(base0 + c * chunk, chunk)],
                    sem_w[s2]),
                pltpu.async_copy(pg_v[s2], acc_sh.at[dst_v[s4]], sem_s[s2],
                                 add=True),
            )

        for d in w_d.pop(kk - 1):
            d.wait()
        plsc.subcore_barrier()
        pltpu.sync_copy(acc_sh.at[rr], zb_v)
        pltpu.sync_copy(zb_v, agg_hbm.at[cid, rr])

    return k


def _make_sc_spmm(n: int, e: int, d: int, chunk: int):
    ew = e // NW
    nrows = n // NS
    kk = ew // chunk
    nz = nrows // chunk
    mesh = plsc.VectorSubcoreMesh(
        core_axis_name="c", subcore_axis_name="s", num_cores=NC,
        num_subcores=NS)

    @functools.partial(
        pl.kernel,
        out_type=jax.ShapeDtypeStruct((NC, n, d), jnp.float32),
        mesh=mesh,
        scratch_types=[
            [pltpu.VMEM((chunk,), jnp.int32) for _ in range(4)],
            [pltpu.VMEM((chunk,), jnp.int32) for _ in range(4)],
            [pltpu.VMEM((chunk,), jnp.float32) for _ in range(4)],
            [pltpu.VMEM((chunk, d), jnp.float32) for _ in range(2)],
            pltpu.VMEM_SHARED((n, d), jnp.float32),
            [pltpu.SemaphoreType.DMA for _ in range(4)],
            [pltpu.SemaphoreType.DMA for _ in range(2)],
            [pltpu.SemaphoreType.DMA for _ in range(2)],
        ],
        compiler_params=pltpu.CompilerParams(use_tc_tiling_on_sc=False,
                                             needs_layout_passes=False),
    )
    def k(hx_hbm, cols_hbm, rows_hbm, vals_hbm, out_hbm,
          cols_v, rows_v, vals_v, g_v, acc_sh, sem_i, sem_g, sem_s):
        cid = lax.axis_index("c")
        sid = lax.axis_index("s")
        wid = sid * NC + cid

        @pl.loop(0, chunk)
        def _(i):
            for j in range(d // 16):
                g_v[0][i, pl.ds(j * 16, 16)] = jnp.zeros((16,), jnp.float32)

        for kz in range(nz):
            pltpu.sync_copy(
                g_v[0], acc_sh.at[pl.ds(sid * nrows + kz * chunk, chunk)])
        plsc.subcore_barrier()

        base0 = wid * ew

        def start_idx(c):
            s4 = c % 4
            base = pl.ds(base0 + c * chunk, chunk)
            return (pltpu.async_copy(cols_hbm.at[base], cols_v[s4],
                                     sem_i[s4]),
                    pltpu.async_copy(rows_hbm.at[base], rows_v[s4],
                                     sem_i[s4]),
                    pltpu.async_copy(vals_hbm.at[base], vals_v[s4],
                                     sem_i[s4]))

        def start_gather(c):
            s4, s2 = c % 4, c % 2
            return (pltpu.async_copy(hx_hbm.at[cols_v[s4]], g_v[s2],
                                     sem_g[s2]),)

        idx_d = {}
        g_d = {}
        w_d = {}
        for c in range(kk):
            s4, s2 = c % 4, c % 2
            if c == 0:
                for dd in start_idx(0):
                    dd.wait()
                g_d[0] = start_gather(0)
                for nx in (1, 2, 3):
                    idx_d[nx] = start_idx(nx)
            if c >= 1:
                for dd in w_d.pop(c - 1):
                    dd.wait()
                if c + 3 < kk:
                    idx_d[c + 3] = start_idx(c + 3)
            if c + 1 < kk:
                for dd in idx_d.pop(c + 1):
                    dd.wait()
                g_d[c + 1] = start_gather(c + 1)
            for dd in g_d.pop(c):
                dd.wait()

            @pl.loop(0, chunk)
            def _(i):
                s = plsc.load_gather(vals_v[s4],
                                     [jnp.full((16,), i, jnp.int32)])
                for j in range(d // 16):
                    g_v[s2][i, pl.ds(j * 16, 16)] = (
                        g_v[s2][i, pl.ds(j * 16, 16)] * s)

            w_d[c] = (pltpu.async_copy(g_v[s2], acc_sh.at[rows_v[s4]],
                                       sem_s[s2], add=True),)

        for dd in w_d.pop(kk - 1):
            dd.wait()
        plsc.subcore_barrier()
        for kz in range(nz):
            sl = pl.ds(sid * nrows + kz * chunk, chunk)
            pltpu.sync_copy(acc_sh.at[sl], g_v[0])
            pltpu.sync_copy(g_v[0], out_hbm.at[cid, sl])

    return k


def _tc_pre_node(x_ref, hx_ref, wpq_ref, wnx_ref, bn_ref,
                 p_ref, q_ref, s1_ref, *, d, de):
    x = x_ref[...]
    h = hx_ref[...]
    pq = (jnp.dot(x, wpq_ref[0:d], preferred_element_type=jnp.float32)
          + jnp.dot(h, wpq_ref[d:2 * d], preferred_element_type=jnp.float32))
    p_ref[...] = pq[:, 0:de]
    q_ref[...] = pq[:, de:2 * de]
    s1_ref[...] = (jnp.dot(x, wnx_ref[0:d], preferred_element_type=jnp.float32)
                   + jnp.dot(h, wnx_ref[d:2 * d],
                             preferred_element_type=jnp.float32)
                   + bn_ref[...])


def _tc_edge_pre(ea_ref, he_ref, w1_ref, w2_ref, be_ref, ep_ref):
    ep_ref[...] = (
        jnp.dot(ea_ref[...], w1_ref[...], preferred_element_type=jnp.float32)
        + jnp.dot(he_ref[...], w2_ref[...],
                  preferred_element_type=jnp.float32)
        + be_ref[...])


def _tc_post_node(s1_ref, a0_ref, a1_ref, hx_ref, sp0_ref, sp1_ref, wna_ref,
                  coeff_ref, xo_ref, td_ref, sp_ref):
    agg = a0_ref[...] + a1_ref[...]
    xo = s1_ref[...] + jnp.dot(agg, wna_ref[...],
                               preferred_element_type=jnp.float32)
    xo_ref[...] = xo
    td_ref[...] = xo - hx_ref[...]
    sp_ref[...] = (-coeff_ref[0, 0]) * (sp0_ref[...] + sp1_ref[...])


def kernel(x_seq, edge_attr_seq, h_x, h_e, lap_vals, We, be, Wn, bn, coeff,
           edge_index, lap_rows, lap_cols):
    t_steps, n, d = x_seq.shape
    e, de = edge_attr_seq.shape[1], edge_attr_seq.shape[2]

    src = edge_index[0]
    dst = edge_index[1]
    er = e // 8
    wpq = jnp.concatenate([We[0:2 * d], We[2 * d:4 * d]], axis=1)
    eye8 = jnp.eye(8, dtype=jnp.float32)
    wee1 = jnp.kron(eye8, We[4 * d:4 * d + de])
    wee2 = jnp.kron(eye8, We[4 * d + de:])
    wnx = Wn[0:2 * d]
    wna = Wn[2 * d:]
    be_p = jnp.tile(be, 8).reshape(1, 8 * de)
    bn2 = bn.reshape(1, d)
    coeff2 = jnp.reshape(coeff, (1, 1))

    bn_blk = 2000
    be_blk = 16000

    pre_node = pl.pallas_call(
        functools.partial(_tc_pre_node, d=d, de=de),
        grid=(n // bn_blk,),
        in_specs=[
            pl.BlockSpec((bn_blk, d), lambda i: (i, 0)),
            pl.BlockSpec((bn_blk, d), lambda i: (i, 0)),
            pl.BlockSpec((2 * d, 2 * de), lambda i: (0, 0)),
            pl.BlockSpec((2 * d, d), lambda i: (0, 0)),
            pl.BlockSpec((1, d), lambda i: (0, 0)),
        ],
        out_specs=[
            pl.BlockSpec((bn_blk, de), lambda i: (i, 0)),
            pl.BlockSpec((bn_blk, de), lambda i: (i, 0)),
            pl.BlockSpec((bn_blk, d), lambda i: (i, 0)),
        ],
        out_shape=[
            jax.ShapeDtypeStruct((n, de), jnp.float32),
            jax.ShapeDtypeStruct((n, de), jnp.float32),
            jax.ShapeDtypeStruct((n, d), jnp.float32),
        ],
    )

    ber_blk = be_blk // 8
    edge_pre = pl.pallas_call(
        _tc_edge_pre,
        grid=(er // ber_blk,),
        in_specs=[
            pl.BlockSpec((ber_blk, 8 * de), lambda i: (i, 0)),
            pl.BlockSpec((ber_blk, 8 * de), lambda i: (i, 0)),
            pl.BlockSpec((8 * de, 8 * de), lambda i: (0, 0)),
            pl.BlockSpec((8 * de, 8 * de), lambda i: (0, 0)),
            pl.BlockSpec((1, 8 * de), lambda i: (0, 0)),
        ],
        out_specs=pl.BlockSpec((ber_blk, 8 * de), lambda i: (i, 0)),
        out_shape=jax.ShapeDtypeStruct((er, 8 * de), jnp.float32),
    )

    post_node = pl.pallas_call(
        _tc_post_node,
        grid=(n // bn_blk,),
        in_specs=[
            pl.BlockSpec((bn_blk, d), lambda i: (i, 0)),
            pl.BlockSpec((bn_blk, de), lambda i: (i, 0)),
            pl.BlockSpec((bn_blk, de), lambda i: (i, 0)),
            pl.BlockSpec((bn_blk, d), lambda i: (i, 0)),
            pl.BlockSpec((bn_blk, d), lambda i: (i, 0)),
            pl.BlockSpec((bn_blk, d), lambda i: (i, 0)),
            pl.BlockSpec((de, d), lambda i: (0, 0)),
            pl.BlockSpec(memory_space=pltpu.SMEM),
        ],
        out_specs=[
            pl.BlockSpec((bn_blk, d), lambda i: (i, 0)),
            pl.BlockSpec((bn_blk, d), lambda i: (i, 0)),
            pl.BlockSpec((bn_blk, d), lambda i: (i, 0)),
        ],
        out_shape=[
            jax.ShapeDtypeStruct((n, d), jnp.float32),
            jax.ShapeDtypeStruct((n, d), jnp.float32),
            jax.ShapeDtypeStruct((n, d), jnp.float32),
        ],
    )

    n_pad = ((n + 639) // 640) * 640
    sc_edge = _make_sc_edge(n_pad, e, de, chunk=1000)
    sc_spmm = _make_sc_spmm(n_pad, e, d, chunk=80)

    hx, he = h_x, jnp.reshape(h_e, (er, 8 * de))
    out_x, out_e, tds, sps = [], [], [], []
    for t in range(t_steps):
        p, q, s1 = pre_node(x_seq[t], hx, wpq, wnx, bn2)
        p = jnp.pad(p, ((0, n_pad - n), (0, 0)))
        q = jnp.pad(q, ((0, n_pad - n), (0, 0)))
        epart = edge_pre(jnp.reshape(edge_attr_seq[t], (er, 8 * de)),
                         he, wee1, wee2, be_p)
        e_out, agg2 = sc_edge(p, q, epart, src, dst)
        sp2 = sc_spmm(hx, lap_cols, lap_rows, lap_vals)
        x_out, td, sp = post_node(s1, agg2[0, :n], agg2[1, :n], hx,
                                  sp2[0, :n], sp2[1, :n], wna, coeff2)
        hx, he = x_out, jnp.reshape(e_out, (er, 8 * de))
        out_x.append(x_out)
        out_e.append(e_out)
        tds.append(td)
        sps.append(sp)

    return (jnp.stack(out_x), jnp.stack(out_e), jnp.stack(tds),
            jnp.stack(sps))
